# baseline (speedup 1.0000x reference)
import functools

import jax
import jax.numpy as jnp
from jax import lax
from jax.experimental import pallas as pl
from jax.experimental.pallas import tpu as pltpu
from jax.experimental.pallas import tpu_sc as plsc

B = 16384
L = 20
N = B * L
RAD_D = 32
POS_D = 128
OUT_D = RAD_D + POS_D

NC = 2
NS = 16
NW = NC * NS
N_PER_W = N // NW
G = 128
C = 640
CB = C // L
N_ITERS = N_PER_W // C
B_PER_W = B // NW


def _renorm_rows(x):
    ss = jnp.sum(x * x, axis=-1, keepdims=True)
    norm = jnp.sqrt(ss)
    scale = jnp.minimum(1.0, 1.0 / jnp.maximum(norm, 1e-7))
    return x * scale


def _renorm_kernel(tab_ref, out_ref):
    out_ref[...] = _renorm_rows(tab_ref[...])


def _prescale_rad(table):
    blk = 2048
    return pl.pallas_call(
        _renorm_kernel,
        grid=(pl.cdiv(table.shape[0], blk),),
        in_specs=[pl.BlockSpec((blk, RAD_D), lambda i: (i, 0))],
        out_specs=pl.BlockSpec((blk, RAD_D), lambda i: (i, 0)),
        out_shape=jax.ShapeDtypeStruct(table.shape, table.dtype),
    )(table)


def _prescale_pos(table):
    return pl.pallas_call(
        _renorm_kernel,
        out_shape=jax.ShapeDtypeStruct(table.shape, table.dtype),
    )(table)


def _sc_gather_concat(rad_tab, pos_tab, rad_idx2d, pos_idx2d):
    mesh = plsc.VectorSubcoreMesh(core_axis_name="c", subcore_axis_name="s")

    @functools.partial(
        pl.kernel,
        out_type=jax.ShapeDtypeStruct((B, L, OUT_D), jnp.float32),
        mesh=mesh,
        compiler_params=pltpu.CompilerParams(use_tc_tiling_on_sc=False),
        scratch_types=[
            pltpu.VMEM((C // G, G), jnp.int32),
            pltpu.VMEM((C // G, G), jnp.int32),
            pltpu.VMEM((C, RAD_D), jnp.float32),
            pltpu.VMEM((C, POS_D), jnp.float32),
            pltpu.SemaphoreType.DMA,
        ],
    )
    def k(rad_tab_hbm, pos_tab_hbm, ridx_hbm, pidx_hbm, out3d_hbm,
          ridx_v, pidx_v, rrows_v, prows_v, sem):
        wid = lax.axis_index("s") * NC + lax.axis_index("c")
        row0 = wid * (N_PER_W // G)
        b0 = wid * B_PER_W

        def body(j, carry):
            irow = row0 + j * (C // G)
            pltpu.sync_copy(ridx_hbm.at[pl.ds(irow, C // G)], ridx_v)
            pltpu.sync_copy(pidx_hbm.at[pl.ds(irow, C // G)], pidx_v)
            copies = []
            for kk in range(C // G):
                copies.append(pltpu.async_copy(
                    rad_tab_hbm.at[ridx_v.at[kk]],
                    rrows_v.at[pl.ds(kk * G, G)], sem))
                copies.append(pltpu.async_copy(
                    pos_tab_hbm.at[pidx_v.at[kk]],
                    prows_v.at[pl.ds(kk * G, G)], sem))
            for cp in copies:
                cp.wait()
            ob = b0 + j * CB
            for i in range(CB):
                pltpu.sync_copy(rrows_v.at[pl.ds(i * L, L)],
                                out3d_hbm.at[ob + i, :, pl.ds(0, RAD_D)])
                pltpu.sync_copy(prows_v.at[pl.ds(i * L, L)],
                                out3d_hbm.at[ob + i, :, pl.ds(RAD_D, POS_D)])
            return carry

        lax.fori_loop(0, N_ITERS, body, 0, unroll=False)

    return k(rad_tab, pos_tab, rad_idx2d, pos_idx2d)


def kernel(radical_indices, position_labels, rademb_table, posemb_table):
    rad_tab = _prescale_rad(rademb_table)
    pos_tab = _prescale_pos(posemb_table)
    ridx = radical_indices.reshape(N // G, G).astype(jnp.int32)
    pidx = position_labels.reshape(N // G, G).astype(jnp.int32)
    return _sc_gather_concat(rad_tab, pos_tab, ridx, pidx)


# speedup vs baseline: 3.3222x; 3.3222x over previous
import functools

import jax
import jax.numpy as jnp
from jax import lax
from jax.experimental import pallas as pl
from jax.experimental.pallas import tpu as pltpu
from jax.experimental.pallas import tpu_sc as plsc

B = 16384
L = 20
N = B * L
RAD_D = 32
POS_D = 128
OUT_D = RAD_D + POS_D
POS_V = 512

NC = 2
NS = 16
NW = NC * NS
BG = B // 128
Q = BG // NW
D2G = OUT_D // 8
T_STEPS = L * Q


def _renorm_rows(x):
    ss = jnp.sum(x * x, axis=-1, keepdims=True)
    norm = jnp.sqrt(ss)
    scale = jnp.minimum(1.0, 1.0 / jnp.maximum(norm, 1e-7))
    return x * scale


def _rad_scale_kernel_t(tab_ref, out_ref):
    x = tab_ref[...]
    ss = jnp.sum(x * x, axis=0, keepdims=True)
    norm = jnp.sqrt(ss)
    out_ref[...] = jnp.minimum(1.0, 1.0 / jnp.maximum(norm, 1e-7))


def _rad_scales(table_t):
    blk = 4096
    r = table_t.shape[1]
    return pl.pallas_call(
        _rad_scale_kernel_t,
        grid=(pl.cdiv(r, blk),),
        in_specs=[pl.BlockSpec((RAD_D, blk), lambda i: (0, i))],
        out_specs=pl.BlockSpec((1, blk), lambda i: (0, i)),
        out_shape=jax.ShapeDtypeStruct((1, r), jnp.float32),
    )(table_t)


def _pos_renorm_t_kernel(tab_ref, out_ref):
    y = _renorm_rows(tab_ref[...])
    out_ref[...] = y[:POS_V, :].T


def _prescale_pos_t(table):
    return pl.pallas_call(
        _pos_renorm_t_kernel,
        out_shape=jax.ShapeDtypeStruct((POS_D, POS_V), table.dtype),
    )(table)


def _sc_gather_t(rad_tab, rad_scales, pos_tab_t, ridx3, pidx3):
    mesh = plsc.VectorSubcoreMesh(core_axis_name="c", subcore_axis_name="s")

    @functools.partial(
        pl.kernel,
        out_type=jax.ShapeDtypeStruct((L, D2G, BG, 8, 128), jnp.float32),
        mesh=mesh,
        compiler_params=pltpu.CompilerParams(
            use_tc_tiling_on_sc=False, needs_layout_passes=False),
        scratch_types=[
            pltpu.VMEM((POS_D, POS_V), jnp.float32),
            pltpu.VMEM((2, Q, 128), jnp.int32),
            pltpu.VMEM((2, Q, 128), jnp.int32),
            pltpu.VMEM((2, 128, RAD_D), jnp.float32),
            pltpu.VMEM((2, 128), jnp.float32),
            pltpu.VMEM((2, D2G, 8, 128), jnp.float32),
            pltpu.SemaphoreType.DMA,
            pltpu.SemaphoreType.DMA,
            pltpu.SemaphoreType.DMA,
            pltpu.SemaphoreType.DMA,
        ],
    )
    def k(rad_tab_hbm, rscale_hbm, pos_t_hbm, ridx_hbm, pidx_hbm, out5_hbm,
          pos_v, ridx_v, pidx_v, radrows_v, rscales_v, outb_v,
          gsem, ssem, wsem, isem):
        wid = lax.axis_index("s") * NC + lax.axis_index("c")
        g0 = wid * Q

        pltpu.sync_copy(pos_t_hbm, pos_v)
        pltpu.sync_copy(ridx_hbm.at[0, pl.ds(g0, Q)], ridx_v.at[0])
        pltpu.sync_copy(pidx_hbm.at[0, pl.ds(g0, Q)], pidx_v.at[0])

        lane_ids = [lax.iota(jnp.int32, 16) + (16 * g) for g in range(8)]
        d2_sp = [jnp.full((16,), d2, jnp.int32) for d2 in range(RAD_D)]

        def issue_idx(d1n):
            sl = d1n % 2
            pltpu.async_copy(ridx_hbm.at[d1n, pl.ds(g0, Q)], ridx_v.at[sl], isem)
            pltpu.async_copy(pidx_hbm.at[d1n, pl.ds(g0, Q)], pidx_v.at[sl], isem)

        def drain_idx():
            pltpu.make_async_copy(
                ridx_hbm.at[0, pl.ds(g0, Q)], ridx_v.at[0], isem).wait()
            pltpu.make_async_copy(
                pidx_hbm.at[0, pl.ds(g0, Q)], pidx_v.at[0], isem).wait()

        def issue_rad(t, slot):
            d1 = t // Q
            q = t % Q
            pltpu.async_copy(
                rad_tab_hbm.at[ridx_v.at[d1 % 2, q]], radrows_v.at[slot], gsem)
            pltpu.async_copy(
                rscale_hbm.at[ridx_v.at[d1 % 2, q]], rscales_v.at[slot], ssem)

        def drain_rad(slot):
            pltpu.make_async_copy(
                rad_tab_hbm.at[pl.ds(0, 128)], radrows_v.at[slot], gsem).wait()
            pltpu.make_async_copy(
                rscale_hbm.at[pl.ds(0, 128)], rscales_v.at[slot], ssem).wait()

        def drain_write(slot):
            pltpu.make_async_copy(
                outb_v.at[slot], out5_hbm.at[0, :, 0], wsem).wait()

        issue_rad(0, 0)

        def step(t, carry):
            d1 = t // Q
            q = t % Q
            slot = t % 2

            @pl.when(jnp.logical_and(q == 0, d1 + 1 < L))
            def _():
                issue_idx(d1 + 1)

            @pl.when(t + 1 < T_STEPS)
            def _():
                @pl.when(q == Q - 1)
                def _():
                    drain_idx()
                issue_rad(t + 1, 1 - slot)

            drain_rad(slot)

            @pl.when(t >= 2)
            def _():
                drain_write(slot)

            idx_g = [pidx_v[d1 % 2, q, pl.ds(16 * g, 16)] for g in range(8)]
            rad2d = radrows_v.at[slot]
            outb = outb_v.at[slot]

            @plsc.parallel_loop(0, POS_D, unroll=2)
            def pos_row(d2):
                sp = jnp.full((16,), d2, jnp.int32)
                row = RAD_D + d2
                for g in range(8):
                    outb[row // 8, row % 8, pl.ds(16 * g, 16)] = plsc.load_gather(
                        pos_v, [sp, idx_g[g]])

            scales_g = [rscales_v[slot, pl.ds(16 * g, 16)] for g in range(8)]

            @plsc.parallel_loop(0, RAD_D, unroll=4)
            def rad_row(d2):
                sp = jnp.full((16,), d2, jnp.int32)
                for g in range(8):
                    outb[d2 // 8, d2 % 8, pl.ds(16 * g, 16)] = scales_g[g] * \
                        plsc.load_gather(rad2d, [lane_ids[g], sp])

            pltpu.async_copy(outb, out5_hbm.at[d1, :, g0 + q], wsem)
            return carry

        lax.fori_loop(0, T_STEPS, step, 0, unroll=False)

        drain_write(0)
        drain_write(1)

    return k(rad_tab, rad_scales, pos_tab_t, ridx3, pidx3)


def kernel(radical_indices, position_labels, rademb_table, posemb_table):
    rad_tab = rademb_table.astype(jnp.float32)
    rad_sc = _rad_scales(rademb_table.T).reshape(rademb_table.shape[0])
    pos_tab_t = _prescale_pos_t(posemb_table)
    ridx3 = radical_indices.T.reshape(L, BG, 128).astype(jnp.int32)
    pidx3 = position_labels.T.reshape(L, BG, 128).astype(jnp.int32)
    out5 = _sc_gather_t(rad_tab, rad_sc, pos_tab_t, ridx3, pidx3)
    return out5.transpose(2, 4, 0, 1, 3).reshape(B, L, OUT_D)
